# Initial kernel scaffold; baseline (speedup 1.0000x reference)
import jax
import jax.numpy as jnp
from jax.experimental import pallas as pl


def kernel(x, hyperedge_index, W_conv, b_conv, W_read, b_read):
    raise NotImplementedError("write your pallas kernel here")


# speedup vs baseline: 49.4417x; 49.4417x over previous
import functools

import jax
import jax.numpy as jnp
from jax import lax
from jax.experimental import pallas as pl
from jax.experimental.pallas import tpu as pltpu
from jax.experimental.pallas import tpu_sc as plsc

NC = 2
NS = 16
NW = NC * NS


def _row_map(i):
    return (i, jnp.int32(0))


def _zero_map(i):
    return (jnp.int32(0), jnp.int32(0))


def _zero_map1(i):
    return (jnp.int32(0),)
K = 128


def _mesh():
    return plsc.VectorSubcoreMesh(core_axis_name="c", subcore_axis_name="s",
                                  num_cores=NC, num_subcores=NS)


def _fori(n, body):
    lax.fori_loop(jnp.int32(0), jnp.int32(n), body, jnp.int32(0))


def _zero_vmem_2d(ref, rows, cols):
    z = jnp.zeros((16,), jnp.float32)

    def body(i, _):
        r = i // (cols // 16)
        c = i % (cols // 16)
        ref[r, pl.ds(c * 16, 16)] = z
        return _

    _fori(rows * (cols // 16), body)


def _zero_vmem_1d(ref, n):
    z = jnp.zeros((16,), jnp.float32)

    def body(i, _):
        ref[pl.ds(i * 16, 16)] = z
        return _

    _fori(n // 16, body)


def _make_degree_kernel(total, R, S):
    CH = S // K

    @functools.partial(
        pl.kernel,
        out_type=(
            jax.ShapeDtypeStruct((NC, R), jnp.float32),
            jax.ShapeDtypeStruct((NC, R), jnp.float32),
        ),
        mesh=_mesh(),
        scratch_types=[
            pltpu.VMEM((K,), jnp.int32),
            pltpu.VMEM((K,), jnp.int32),
            pltpu.VMEM((K,), jnp.float32),
            pltpu.VMEM((K,), jnp.float32),
            pltpu.VMEM_SHARED((R,), jnp.float32),
            pltpu.VMEM_SHARED((R,), jnp.float32),
        ],
    )
    def deg_kernel(ni_hbm, ei_hbm, degv_out, dege_out,
                   idxn_v, idxe_v, ones_v, zero_v, dv_acc, de_acc):
        c = lax.axis_index("c")
        s = lax.axis_index("s")
        wid = c * NS + s
        stripe = R // NS

        one = jnp.ones((16,), jnp.float32)

        def ones_body(i, _):
            ones_v[pl.ds(i * 16, 16)] = one
            return _

        _fori(K // 16, ones_body)
        _zero_vmem_1d(zero_v, K)

        def zcopy(i, _):
            pltpu.sync_copy(zero_v, dv_acc.at[pl.ds(s * stripe + i * K, K)])
            pltpu.sync_copy(zero_v, de_acc.at[pl.ds(s * stripe + i * K, K)])
            return _

        _fori(stripe // K, zcopy)
        plsc.subcore_barrier()

        def chunk(j, _):
            base = wid * S + j * K
            pltpu.sync_copy(ni_hbm.at[pl.ds(base, K)], idxn_v)
            pltpu.sync_copy(ei_hbm.at[pl.ds(base, K)], idxe_v)
            pltpu.sync_copy(ones_v, dv_acc.at[idxn_v], add=True)
            pltpu.sync_copy(ones_v, de_acc.at[idxe_v], add=True)
            return _

        _fori(CH, chunk)
        plsc.subcore_barrier()

        pltpu.sync_copy(dv_acc.at[pl.ds(s * stripe, stripe)],
                        degv_out.at[c, pl.ds(s * stripe, stripe)])
        pltpu.sync_copy(de_acc.at[pl.ds(s * stripe, stripe)],
                        dege_out.at[c, pl.ds(s * stripe, stripe)])

    return deg_kernel


def _make_row_pass_kernel(total, R, S, D, with_scalar):
    CH = S // K
    out_types = [jax.ShapeDtypeStruct((NC, R, D), jnp.float32)]
    scratch = [
        pltpu.VMEM((K,), jnp.int32),
        pltpu.VMEM((K,), jnp.int32),
        pltpu.VMEM((K, D), jnp.float32),
        pltpu.VMEM((K,), jnp.float32),
        pltpu.VMEM_SHARED((R, D), jnp.float32),
        pltpu.VMEM_SHARED((R,), jnp.float32),
    ]
    if with_scalar:
        out_types.append(jax.ShapeDtypeStruct((NC, R), jnp.float32))

    def body(*refs):
        if with_scalar:
            (tab_hbm, stab_hbm, gi_hbm, si_hbm, rows_out, s_out,
             gidx_v, sidx_v, rows_v, srow_v, acc, s_acc) = refs
        else:
            (tab_hbm, gi_hbm, si_hbm, rows_out,
             gidx_v, sidx_v, rows_v, srow_v, acc, s_acc) = refs
        c = lax.axis_index("c")
        s = lax.axis_index("s")
        wid = c * NS + s
        stripe = R // NS

        _zero_vmem_2d(rows_v, K, D)
        _zero_vmem_1d(srow_v, K)

        def zcopy(i, _):
            pltpu.sync_copy(rows_v, acc.at[pl.ds(s * stripe + i * K, K)])
            pltpu.sync_copy(srow_v, s_acc.at[pl.ds(s * stripe + i * K, K)])
            return _

        _fori(stripe // K, zcopy)
        plsc.subcore_barrier()

        def chunk(j, _):
            base = wid * S + j * K
            pltpu.sync_copy(gi_hbm.at[pl.ds(base, K)], gidx_v)
            pltpu.sync_copy(si_hbm.at[pl.ds(base, K)], sidx_v)
            pltpu.sync_copy(tab_hbm.at[gidx_v], rows_v)
            pltpu.sync_copy(rows_v, acc.at[sidx_v], add=True)
            if with_scalar:
                pltpu.sync_copy(stab_hbm.at[gidx_v], srow_v)
                pltpu.sync_copy(srow_v, s_acc.at[sidx_v], add=True)
            return _

        _fori(CH, chunk)
        plsc.subcore_barrier()

        pltpu.sync_copy(acc.at[pl.ds(s * stripe, stripe)],
                        rows_out.at[c, pl.ds(s * stripe, stripe)])
        if with_scalar:
            pltpu.sync_copy(s_acc.at[pl.ds(s * stripe, stripe)],
                            s_out.at[c, pl.ds(s * stripe, stripe)])

    return pl.kernel(
        body,
        out_type=tuple(out_types) if with_scalar else out_types[0],
        mesh=_mesh(),
        scratch_types=scratch,
    )


def _edge_linear(he0, he1, dvs0, dvs1, de0, de1, W, b):
    R = he0.shape[0]
    BLK = 512
    grid = (R // BLK,)

    def body(he0_r, he1_r, dvs0_r, dvs1_r, de0_r, de1_r, W_r, b_r, out_r):
        dege = jnp.maximum(de0_r[...] + de1_r[...], 1.0)
        hesum = he0_r[...] + he1_r[...]
        dvs = dvs0_r[...] + dvs1_r[...]
        det = jnp.maximum(dvs / dege, 1.0)
        he = hesum * (1.0 / dege)[:, None]
        hel = jnp.dot(he, W_r[...], preferred_element_type=jnp.float32) + b_r[...][None, :]
        out_r[...] = hel * lax.rsqrt(det)[:, None]

    return pl.pallas_call(
        body,
        grid=grid,
        in_specs=[
            pl.BlockSpec((BLK, 128), _row_map),
            pl.BlockSpec((BLK, 128), _row_map),
            pl.BlockSpec((BLK,), lambda i: (i,)),
            pl.BlockSpec((BLK,), lambda i: (i,)),
            pl.BlockSpec((BLK,), lambda i: (i,)),
            pl.BlockSpec((BLK,), lambda i: (i,)),
            pl.BlockSpec((128, 128), _zero_map),
            pl.BlockSpec((128,), _zero_map1),
        ],
        out_specs=pl.BlockSpec((BLK, 128), _row_map),
        out_shape=jax.ShapeDtypeStruct((R, 128), jnp.float32),
    )(he0, he1, dvs0, dvs1, de0, de1, W, b)


def _readout(agg0, agg1, dvt, Wr, br):
    R = agg0.shape[0]
    BLK = 512
    grid = (R // BLK,)

    def body(a0_r, a1_r, dv_r, wr_r, br_r, out_r):
        agg = a0_r[...] + a1_r[...]
        dv = jnp.maximum(dv_r[...], 1.0)
        h = jnp.maximum(agg * lax.rsqrt(dv)[:, None], 0.0)
        y = jnp.sum(h * wr_r[...], axis=-1) + br_r[0]
        out_r[...] = y

    return pl.pallas_call(
        body,
        grid=grid,
        in_specs=[
            pl.BlockSpec((BLK, 128), _row_map),
            pl.BlockSpec((BLK, 128), _row_map),
            pl.BlockSpec((BLK,), lambda i: (i,)),
            pl.BlockSpec((1, 128), _zero_map),
            pl.BlockSpec((1,), _zero_map1, memory_space=pltpu.SMEM),
        ],
        out_specs=pl.BlockSpec((BLK,), lambda i: (i,)),
        out_shape=jax.ShapeDtypeStruct((R,), jnp.float32),
    )(agg0, agg1, dvt, Wr, br)


def kernel(x, hyperedge_index, W_conv, b_conv, W_read, b_read):
    N, D = x.shape
    NNZ = hyperedge_index.shape[1]
    out_dtype = jnp.result_type(x.dtype, W_conv.dtype, W_read.dtype)
    x = x.astype(jnp.float32)
    W_conv = W_conv.astype(jnp.float32)
    b_conv = b_conv.astype(jnp.float32)
    W_read = W_read.astype(jnp.float32)
    b_read = b_read.astype(jnp.float32)

    span = K * NW
    total = ((NNZ + span - 1) // span) * span
    S = total // NW
    pad = total - NNZ
    trash = N
    R = ((N + 1 + (NS * K) - 1) // (NS * K)) * (NS * K)

    ni = hyperedge_index[0].astype(jnp.int32)
    ei = hyperedge_index[1].astype(jnp.int32)
    zpad = jnp.zeros((pad,), jnp.int32)
    tpad = jnp.full((pad,), trash, jnp.int32)
    ni_g = jnp.concatenate([ni, zpad])
    ni_s = jnp.concatenate([ni, tpad])
    ei_g = jnp.concatenate([ei, zpad])
    ei_s = jnp.concatenate([ei, tpad])

    deg = _make_degree_kernel(total, R, S)
    degv_part, dege_part = deg(ni_s, ei_s)
    dv_tab = degv_part[0] + degv_part[1]


    rowpass = _make_row_pass_kernel(total, R, S, D, with_scalar=True)
    he_part, dvs_part = rowpass(x, dv_tab, ni_g, ei_s)


    he_s = _edge_linear(he_part[0], he_part[1], dvs_part[0], dvs_part[1],
                        dege_part[0], dege_part[1], W_conv, b_conv)

    aggpass = _make_row_pass_kernel(total, R, S, D, with_scalar=False)
    agg_part = aggpass(he_s, ei_g, ni_s)

    y = _readout(agg_part[0], agg_part[1], dv_tab, W_read, b_read)
    return y[:N].astype(out_dtype)
